# baseline (speedup 1.0000x reference)
import functools

import jax
import jax.numpy as jnp
from jax.experimental import pallas as pl
from jax.experimental.pallas import tpu as pltpu

NUM_EXPERTS = 8
TOP_K = 2
HIDDEN = 1024
FFN = 4096
FBLK = 1024
NF = FFN // FBLK
NBUF = 3


def _moe_kernel(x_ref, gate_ref, w1_hbm, w3_hbm, w2_hbm, out_ref,
                wmat_ref, cnt_ref):
    x = x_ref[...]
    logits = jnp.dot(x, gate_ref[...].T, preferred_element_type=jnp.float32)
    p = jax.nn.softmax(logits, axis=-1)
    cols = jax.lax.broadcasted_iota(jnp.int32, p.shape, 1)
    i1 = jnp.argmax(p, axis=-1)
    oh1 = (cols == i1[:, None])
    m1 = jnp.max(p, axis=-1, keepdims=True)
    p2 = jnp.where(oh1, -jnp.inf, p)
    i2 = jnp.argmax(p2, axis=-1)
    oh2 = (cols == i2[:, None])
    m2 = jnp.max(p2, axis=-1, keepdims=True)
    s = m1 + m2
    wmat_ref[...] = oh1 * (m1 / s) + oh2 * (m2 / s)
    out_ref[...] = jnp.zeros_like(out_ref)
    cnt_ref[0] = 0

    xb = x.astype(jnp.bfloat16)

    def body(w1_ref, w3_ref, w2_ref):
        step = cnt_ref[0]
        cnt_ref[0] = step + 1
        e = step // NF
        w1b = w1_ref[0].astype(jnp.bfloat16)
        w3b = w3_ref[0].astype(jnp.bfloat16)
        h1 = jnp.dot(xb, w1b.T, preferred_element_type=jnp.float32)
        h3 = jnp.dot(xb, w3b.T, preferred_element_type=jnp.float32)
        eoh = (jax.lax.broadcasted_iota(jnp.int32, (NUM_EXPERTS, 1), 0) == e)
        wcol = jnp.dot(wmat_ref[...], eoh.astype(jnp.float32),
                       preferred_element_type=jnp.float32)
        h = (jax.nn.silu(h1) * h3 * wcol).astype(jnp.bfloat16)
        w2b = w2_ref[0].astype(jnp.bfloat16)
        out_ref[...] += jnp.dot(h, w2b.T, preferred_element_type=jnp.float32)

    buffered = pl.Buffered(buffer_count=NBUF)
    pipeline = pltpu.emit_pipeline(
        body,
        grid=(NUM_EXPERTS, NF),
        in_specs=[
            pl.BlockSpec((1, FBLK, HIDDEN), lambda e, f: (e, f, 0),
                         pipeline_mode=buffered),
            pl.BlockSpec((1, FBLK, HIDDEN), lambda e, f: (e, f, 0),
                         pipeline_mode=buffered),
            pl.BlockSpec((1, HIDDEN, FBLK), lambda e, f: (e, 0, f),
                         pipeline_mode=buffered),
        ],
    )
    pipeline(w1_hbm, w3_hbm, w2_hbm)


@functools.partial(jax.jit, static_argnames=())
def kernel(hidden_states, gate_w, w1, w2, w3):
    b, s, d = hidden_states.shape
    x = hidden_states.reshape(-1, d)
    t = x.shape[0]

    out = pl.pallas_call(
        _moe_kernel,
        in_specs=[
            pl.BlockSpec((t, HIDDEN), lambda: (0, 0)),
            pl.BlockSpec((NUM_EXPERTS, HIDDEN), lambda: (0, 0)),
            pl.BlockSpec(memory_space=pl.ANY),
            pl.BlockSpec(memory_space=pl.ANY),
            pl.BlockSpec(memory_space=pl.ANY),
        ],
        out_specs=pl.BlockSpec((t, HIDDEN), lambda: (0, 0)),
        out_shape=jax.ShapeDtypeStruct((t, HIDDEN), jnp.float32),
        scratch_shapes=[
            pltpu.VMEM((t, NUM_EXPERTS), jnp.float32),
            pltpu.SMEM((1,), jnp.int32),
        ],
    )(x, gate_w, w1, w3, w2)
    return out.reshape(b, s, d)
